# Initial kernel scaffold; baseline (speedup 1.0000x reference)
import jax
import jax.numpy as jnp
from jax.experimental import pallas as pl


def kernel(state, state_buffer, W1, b1, W2, b2, W3, b3):
    raise NotImplementedError("write your pallas kernel here")


# speedup vs baseline: 17.1727x; 17.1727x over previous
import functools

import jax
import jax.numpy as jnp
from jax.experimental import pallas as pl
from jax.experimental.pallas import tpu as pltpu

Q_TILE = 128
N_TILE = 2048
N_PAD = 20480
N_REAL = 20000
BIG = 1.0e30
K_LIST = (10, 15, 20)


def _novelty_kernel(state_ref, buf_ref, w1_ref, b1_ref, w2_ref, b2_ref,
                    w3_ref, b3_ref, out_ref, d2_scratch):
    j = pl.program_id(1)
    n_j = pl.num_programs(1)

    s = state_ref[...]
    b = buf_ref[...]

    prod = jax.lax.dot_general(s, b, (((1,), (1,)), ((), ())),
                               preferred_element_type=jnp.float32)
    ones = jnp.ones((1, b.shape[1]), jnp.float32)
    bn = jax.lax.dot_general(ones, b * b, (((1,), (1,)), ((), ())),
                             preferred_element_type=jnp.float32)
    d2 = bn - 2.0 * prod

    col = jax.lax.broadcasted_iota(jnp.int32, (Q_TILE, N_TILE), 1) + j * N_TILE
    d2 = jnp.where(col < N_REAL, d2, BIG)

    d2_scratch[j] = d2

    @pl.when(j == n_j - 1)
    def _finalize():
        qn = jnp.sum(s * s, axis=1, keepdims=True)

        prev = jnp.full((Q_TILE, 1), -BIG, dtype=jnp.float32)
        cnt = jnp.zeros((Q_TILE, 1), dtype=jnp.float32)
        sums = [jnp.zeros((Q_TILE, 1), dtype=jnp.float32) for _ in K_LIST]

        def body(_, carry):
            prev, cnt, s10, s15, s20 = carry

            m = jnp.full((Q_TILE, 1), BIG, jnp.float32)
            for cidx in range(N_PAD // N_TILE):
                Dc = d2_scratch[cidx]
                masked = jnp.where(Dc > prev, Dc, BIG)
                m = jnp.minimum(m, jnp.min(masked, axis=1, keepdims=True))

            c = jnp.zeros((Q_TILE, 1), jnp.float32)
            for cidx in range(N_PAD // N_TILE):
                Dc = d2_scratch[cidx]
                c = c + jnp.sum((Dc == m).astype(jnp.float32), axis=1,
                                keepdims=True)

            r = jnp.sqrt(jnp.maximum(m + qn, 1e-12))
            s10 = s10 + r * jnp.clip(10.0 - cnt, 0.0, c)
            s15 = s15 + r * jnp.clip(15.0 - cnt, 0.0, c)
            s20 = s20 + r * jnp.clip(20.0 - cnt, 0.0, c)
            return m, cnt + c, s10, s15, s20

        prev, cnt, s10, s15, s20 = jax.lax.fori_loop(
            0, 20, body, (prev, cnt, *sums))
        ens = (s10 / 10.0 + s15 / 15.0 + s20 / 20.0) / 3.0

        h1 = jnp.maximum(
            jax.lax.dot_general(s, w1_ref[...], (((1,), (0,)), ((), ())),
                                preferred_element_type=jnp.float32)
            + b1_ref[...], 0.0)
        h2 = jnp.maximum(
            jax.lax.dot_general(h1, w2_ref[...], (((1,), (0,)), ((), ())),
                                preferred_element_type=jnp.float32)
            + b2_ref[...], 0.0)
        z = (jax.lax.dot_general(h2, w3_ref[...], (((1,), (0,)), ((), ())),
                                 preferred_element_type=jnp.float32)
             + b3_ref[...])
        logit = z[:, 0:1]
        neural = 1.0 - jax.nn.sigmoid(logit)

        out_ref[...] = (0.7 * ens + 0.3 * neural)[:, 0]


@jax.jit
def kernel(state, state_buffer, W1, b1, W2, b2, W3, b3):
    Q, D = state.shape
    N = state_buffer.shape[0]

    buf = jnp.pad(state_buffer, ((0, N_PAD - N), (0, 0)))
    w1 = W1
    b1p = b1[None, :]
    w2 = jnp.pad(W2, ((0, 0), (0, 128 - W2.shape[1])))
    b2p = jnp.pad(b2, (0, 128 - b2.shape[0]))[None, :]
    w3 = jnp.pad(W3, ((0, 128 - W3.shape[0]), (0, 128 - W3.shape[1])))
    b3p = jnp.pad(b3, (0, 128 - b3.shape[0]))[None, :]

    grid = (Q // Q_TILE, N_PAD // N_TILE)
    out = pl.pallas_call(
        _novelty_kernel,
        grid=grid,
        in_specs=[
            pl.BlockSpec((Q_TILE, D), lambda i, j: (i, 0)),
            pl.BlockSpec((N_TILE, D), lambda i, j: (j, 0)),
            pl.BlockSpec((D, 128), lambda i, j: (0, 0)),
            pl.BlockSpec((1, 128), lambda i, j: (0, 0)),
            pl.BlockSpec((128, 128), lambda i, j: (0, 0)),
            pl.BlockSpec((1, 128), lambda i, j: (0, 0)),
            pl.BlockSpec((128, 128), lambda i, j: (0, 0)),
            pl.BlockSpec((1, 128), lambda i, j: (0, 0)),
        ],
        out_specs=pl.BlockSpec((Q_TILE,), lambda i, j: (i,)),
        out_shape=jax.ShapeDtypeStruct((Q,), jnp.float32),
        scratch_shapes=[pltpu.VMEM((N_PAD // N_TILE, Q_TILE, N_TILE), jnp.float32)],
        compiler_params=pltpu.CompilerParams(
            dimension_semantics=("arbitrary", "arbitrary")),
    )(state, buf, w1, b1p, w2, b2p, w3, b3p)
    return out
